# Initial kernel scaffold; baseline (speedup 1.0000x reference)
import jax
import jax.numpy as jnp
from jax.experimental import pallas as pl


def kernel(inputs, inputsType, emb_freq, emb_w, emb_phase):
    raise NotImplementedError("write your pallas kernel here")


# speedup vs baseline: 6.8118x; 6.8118x over previous
import functools

import jax
import jax.numpy as jnp
from jax import lax
from jax.experimental import pallas as pl
from jax.experimental.pallas import tpu as pltpu
from jax.experimental.pallas import tpu_sc as plsc

PADDING_IDX = 0
WAVE_IDX = -1


def _prep_table(emb_freq, emb_w):
    V, HH = emb_freq.shape
    RB = 2000

    def body(f_ref, w_ref, o_ref):
        o_ref[...] = jnp.concatenate([jnp.exp(f_ref[...]), w_ref[...]], axis=1)

    return pl.pallas_call(
        body,
        grid=(V // RB,),
        in_specs=[
            pl.BlockSpec((RB, HH), lambda i: (i, 0)),
            pl.BlockSpec((RB, HH), lambda i: (i, 0)),
        ],
        out_specs=pl.BlockSpec((RB, 2 * HH), lambda i: (i, 0)),
        out_shape=jax.ShapeDtypeStruct((V, 2 * HH), jnp.float32),
    )(emb_freq, emb_w)


def _sc_gather(table, idx3d):
    NW, CPW, C = idx3d.shape
    D = table.shape[1]
    N = NW * CPW * C
    info = plsc.get_sparse_core_info()
    NC, NS = info.num_cores, info.num_subcores
    assert NW == NC * NS
    mesh = plsc.VectorSubcoreMesh(core_axis_name="c", subcore_axis_name="s")

    @functools.partial(
        pl.kernel,
        mesh=mesh,
        out_type=jax.ShapeDtypeStruct((N, D), jnp.float32),
        scratch_types=[
            pltpu.VMEM((CPW, C), jnp.int32),
            pltpu.VMEM((C, D), jnp.float32),
            pltpu.SemaphoreType.DMA,
        ],
        compiler_params=pltpu.CompilerParams(use_tc_tiling_on_sc=False),
    )
    def k(table_hbm, idx_hbm, out_hbm, idx_v, rows_v, sem):
        wid = lax.axis_index("s") * NC + lax.axis_index("c")
        cbase = wid * CPW
        pltpu.sync_copy(idx_hbm.at[wid], idx_v)

        def body(c, carry):
            pltpu.async_copy(table_hbm.at[idx_v.at[c]], rows_v, sem).wait()
            pltpu.sync_copy(rows_v, out_hbm.at[pl.ds((cbase + c) * C, C)])
            return carry

        lax.fori_loop(0, CPW, body, 0)

    return k(table, idx3d)


def _tc_compute(G3, x_pair, idx_pair, phase64, TB2=256):
    F, BSH, _ = G3.shape
    HH = phase64.shape[2] // 2

    def body(g_ref, x_ref, i_ref, p_ref, o_ref, acc_ref, cnt_ref):
        f = pl.program_id(1)
        nf = pl.num_programs(1)

        @pl.when(f == 0)
        def _():
            acc_ref[...] = jnp.zeros_like(acc_ref)
            cnt_ref[...] = jnp.zeros_like(cnt_ref)

        g = g_ref[0]
        xp = x_ref[0]
        ip = i_ref[0]
        p = p_ref[0]

        valid = jnp.logical_and(ip != PADDING_IDX, ip != WAVE_IDX)
        cnt_ref[...] += valid.astype(jnp.float32)

        ef = jnp.concatenate([g[:, 0:HH], g[:, 2 * HH:3 * HH]], axis=1)
        w = jnp.concatenate([g[:, HH:2 * HH], g[:, 3 * HH:4 * HH]], axis=1)
        xe = jnp.broadcast_to(xp[:, 0:1], (TB2, HH))
        xo = jnp.broadcast_to(xp[:, 1:2], (TB2, HH))
        xb = jnp.concatenate([xe, xo], axis=1)
        d = xb * ef + p
        D = jnp.concatenate([d, d - (jnp.pi / 2)], axis=1)
        w2 = jnp.concatenate([w, w], axis=1)
        acc_ref[...] += jnp.cos(D) * w2

        @pl.when(f == nf - 1)
        def _():
            acc = acc_ref[...]
            vm = lax.rsqrt(2.0 * cnt_ref[...])
            vme = jnp.broadcast_to(vm[:, 0:1], (TB2, HH))
            vmo = jnp.broadcast_to(vm[:, 1:2], (TB2, HH))
            o_ref[...] = jnp.concatenate(
                [
                    acc[:, 0:HH] * vme,
                    acc[:, 2 * HH:3 * HH] * vme,
                    acc[:, HH:2 * HH] * vmo,
                    acc[:, 3 * HH:4 * HH] * vmo,
                ],
                axis=1,
            )

    return pl.pallas_call(
        body,
        grid=(BSH // TB2, F),
        in_specs=[
            pl.BlockSpec((1, TB2, 4 * HH), lambda i, f: (f, i, 0)),
            pl.BlockSpec((1, TB2, 2), lambda i, f: (f, i, 0)),
            pl.BlockSpec((1, TB2, 2), lambda i, f: (f, i, 0)),
            pl.BlockSpec((1, 1, 2 * HH), lambda i, f: (f, 0, 0)),
        ],
        out_specs=pl.BlockSpec((TB2, 4 * HH), lambda i, f: (i, 0)),
        out_shape=jax.ShapeDtypeStruct((BSH, 4 * HH), jnp.float32),
        scratch_shapes=[
            pltpu.VMEM((TB2, 4 * HH), jnp.float32),
            pltpu.VMEM((TB2, 2), jnp.float32),
        ],
    )(G3, x_pair, idx_pair, phase64)


def kernel(inputs, inputsType, emb_freq, emb_w, emb_phase):
    B, S, F = inputs.shape
    V, HH = emb_freq.shape
    BS = B * S
    N = BS * F

    idxT = inputsType.astype(jnp.int32).reshape(BS, F).T
    xT = inputs.reshape(BS, F).T
    idx3d = idxT.reshape(32, N // (32 * 128), 128)
    x_pair = xT.reshape(F, BS // 2, 2)
    idx_pair = idxT.reshape(F, BS // 2, 2)
    phase = emb_phase[1:F + 1]
    phase64 = jnp.concatenate([phase, phase], axis=1).reshape(F, 1, 2 * HH)

    table = _prep_table(emb_freq, emb_w)
    G = _sc_gather(table, idx3d)
    G3 = G.reshape(F, BS // 2, 4 * HH)
    out_pair = _tc_compute(G3, x_pair, idx_pair, phase64)
    return out_pair.reshape(B, S, 2 * HH)


# speedup vs baseline: 7.9518x; 1.1674x over previous
import functools

import jax
import jax.numpy as jnp
from jax import lax
from jax.experimental import pallas as pl
from jax.experimental.pallas import tpu as pltpu
from jax.experimental.pallas import tpu_sc as plsc

PADDING_IDX = 0
WAVE_IDX = -1

_INV2PI = 0.15915494309189535
_COS_C = (
    0.9999999922898433,
    -19.739205553483565,
    64.93917219630283,
    -85.45116501824774,
    60.17622317114795,
    -26.000498056834612,
    6.575565932039546,
)


def _fast_cos(d):
    q = d * _INV2PI
    n = lax.round(q, lax.RoundingMethod.TO_NEAREST_EVEN)
    t = q - n
    u = t * t
    r = jnp.float32(_COS_C[6])
    for c in reversed(_COS_C[:6]):
        r = r * u + jnp.float32(c)
    return r


def _prep_table(emb_freq, emb_w):
    V, HH = emb_freq.shape
    RB = 2000

    def body(f_ref, w_ref, o_ref):
        o_ref[...] = jnp.concatenate([jnp.exp(f_ref[...]), w_ref[...]], axis=1)

    return pl.pallas_call(
        body,
        grid=(V // RB,),
        in_specs=[
            pl.BlockSpec((RB, HH), lambda i: (i, 0)),
            pl.BlockSpec((RB, HH), lambda i: (i, 0)),
        ],
        out_specs=pl.BlockSpec((RB, 2 * HH), lambda i: (i, 0)),
        out_shape=jax.ShapeDtypeStruct((V, 2 * HH), jnp.float32),
    )(emb_freq, emb_w)


def _sc_gather(table, idx_flat):
    D = table.shape[1]
    N = idx_flat.size
    info = plsc.get_sparse_core_info()
    NC, NS = info.num_cores, info.num_subcores
    NW = NC * NS
    C = 128
    CPW = N // (NW * C)
    idx3d = idx_flat.reshape(NW, CPW, C)
    mesh = plsc.VectorSubcoreMesh(core_axis_name="c", subcore_axis_name="s")

    @functools.partial(
        pl.kernel,
        mesh=mesh,
        out_type=jax.ShapeDtypeStruct((N, D), jnp.float32),
        scratch_types=[
            pltpu.VMEM((CPW, C), jnp.int32),
            pltpu.VMEM((C, D), jnp.float32),
            pltpu.SemaphoreType.DMA,
        ],
        compiler_params=pltpu.CompilerParams(use_tc_tiling_on_sc=False),
    )
    def k(table_hbm, idx_hbm, out_hbm, idx_v, rows_v, sem):
        wid = lax.axis_index("s") * NC + lax.axis_index("c")
        cbase = wid * CPW
        pltpu.sync_copy(idx_hbm.at[wid], idx_v)

        def body(c, carry):
            pltpu.async_copy(table_hbm.at[idx_v.at[c]], rows_v, sem).wait()
            pltpu.sync_copy(rows_v, out_hbm.at[pl.ds((cbase + c) * C, C)])
            return carry

        lax.fori_loop(0, CPW, body, 0)

    return k(table, idx3d)


def _tc_compute(G3, x_pair, idx_pair, phase64, TB2=256):
    F, BSH, _ = G3.shape
    HH = phase64.shape[2] // 2

    def body(g_ref, x_ref, i_ref, p_ref, o_ref, acc_ref, cnt_ref):
        f = pl.program_id(1)
        nf = pl.num_programs(1)

        @pl.when(f == 0)
        def _():
            acc_ref[...] = jnp.zeros_like(acc_ref)
            cnt_ref[...] = jnp.zeros_like(cnt_ref)

        g = g_ref[0]
        xp = x_ref[0]
        ip = i_ref[0]
        p = p_ref[0]

        valid = jnp.logical_and(ip != PADDING_IDX, ip != WAVE_IDX)
        cnt_ref[...] += valid.astype(jnp.float32)

        ef = jnp.concatenate([g[:, 0:HH], g[:, 2 * HH:3 * HH]], axis=1)
        w = jnp.concatenate([g[:, HH:2 * HH], g[:, 3 * HH:4 * HH]], axis=1)
        xe = jnp.broadcast_to(xp[:, 0:1], (TB2, HH))
        xo = jnp.broadcast_to(xp[:, 1:2], (TB2, HH))
        xb = jnp.concatenate([xe, xo], axis=1)
        d = xb * ef + p
        D = jnp.concatenate([d, d - (jnp.pi / 2)], axis=1)
        w2 = jnp.concatenate([w, w], axis=1)
        acc_ref[...] += _fast_cos(D) * w2

        @pl.when(f == nf - 1)
        def _():
            acc = acc_ref[...]
            vm = lax.rsqrt(2.0 * cnt_ref[...])
            vme = jnp.broadcast_to(vm[:, 0:1], (TB2, HH))
            vmo = jnp.broadcast_to(vm[:, 1:2], (TB2, HH))
            o_ref[...] = jnp.concatenate(
                [
                    acc[:, 0:HH] * vme,
                    acc[:, 2 * HH:3 * HH] * vme,
                    acc[:, HH:2 * HH] * vmo,
                    acc[:, 3 * HH:4 * HH] * vmo,
                ],
                axis=1,
            )

    return pl.pallas_call(
        body,
        grid=(BSH // TB2, F),
        in_specs=[
            pl.BlockSpec((1, TB2, 4 * HH), lambda i, f: (f, i, 0)),
            pl.BlockSpec((1, TB2, 2), lambda i, f: (f, i, 0)),
            pl.BlockSpec((1, TB2, 2), lambda i, f: (f, i, 0)),
            pl.BlockSpec((1, 1, 2 * HH), lambda i, f: (f, 0, 0)),
        ],
        out_specs=pl.BlockSpec((TB2, 4 * HH), lambda i, f: (i, 0)),
        out_shape=jax.ShapeDtypeStruct((BSH, 4 * HH), jnp.float32),
        scratch_shapes=[
            pltpu.VMEM((TB2, 4 * HH), jnp.float32),
            pltpu.VMEM((TB2, 2), jnp.float32),
        ],
    )(G3, x_pair, idx_pair, phase64)


def kernel(inputs, inputsType, emb_freq, emb_w, emb_phase):
    B, S, F = inputs.shape
    V, HH = emb_freq.shape
    BS = B * S
    N = BS * F

    idxT = inputsType.astype(jnp.int32).reshape(BS, F).T
    xT = inputs.reshape(BS, F).T
    x_pair = xT.reshape(F, BS // 2, 2)
    idx_pair = idxT.reshape(F, BS // 2, 2)
    phase = emb_phase[1:F + 1]
    phase64 = jnp.concatenate([phase, phase], axis=1).reshape(F, 1, 2 * HH)

    table = _prep_table(emb_freq, emb_w)
    G = _sc_gather(table, idxT.reshape(-1))
    G3 = G.reshape(F, BS // 2, 4 * HH)
    out_pair = _tc_compute(G3, x_pair, idx_pair, phase64)
    return out_pair.reshape(B, S, 2 * HH)


# speedup vs baseline: 10.0871x; 1.2685x over previous
import functools

import jax
import jax.numpy as jnp
from jax import lax
from jax.experimental import pallas as pl
from jax.experimental.pallas import tpu as pltpu
from jax.experimental.pallas import tpu_sc as plsc

PADDING_IDX = 0
WAVE_IDX = -1

_INV2PI = 0.15915494309189535
_COS_C = (
    0.9999999922898433,
    -19.739205553483565,
    64.93917219630283,
    -85.45116501824774,
    60.17622317114795,
    -26.000498056834612,
    6.575565932039546,
)


def _fast_cos(d):
    q = d * _INV2PI
    n = lax.round(q, lax.RoundingMethod.TO_NEAREST_EVEN)
    t = q - n
    u = t * t
    r = jnp.float32(_COS_C[6])
    for c in reversed(_COS_C[:6]):
        r = r * u + jnp.float32(c)
    return r


def _prep_table(emb_freq, emb_w):
    V, HH = emb_freq.shape
    RB = 2000

    def body(f_ref, w_ref, o_ref):
        o_ref[...] = jnp.concatenate([jnp.exp(f_ref[...]), w_ref[...]], axis=1)

    return pl.pallas_call(
        body,
        grid=(V // RB,),
        in_specs=[
            pl.BlockSpec((RB, HH), lambda i: (i, 0)),
            pl.BlockSpec((RB, HH), lambda i: (i, 0)),
        ],
        out_specs=pl.BlockSpec((RB, 2 * HH), lambda i: (i, 0)),
        out_shape=jax.ShapeDtypeStruct((V, 2 * HH), jnp.float32),
    )(emb_freq, emb_w)


def _transpose_xi(x, idx, TBt=512):
    BS, F = x.shape

    def body(x_ref, i_ref, xo_ref, io_ref):
        xo_ref[...] = x_ref[...].T
        io_ref[...] = i_ref[...].T

    return pl.pallas_call(
        body,
        grid=(BS // TBt,),
        in_specs=[
            pl.BlockSpec((TBt, F), lambda i: (i, 0)),
            pl.BlockSpec((TBt, F), lambda i: (i, 0)),
        ],
        out_specs=[
            pl.BlockSpec((F, TBt), lambda i: (0, i)),
            pl.BlockSpec((F, TBt), lambda i: (0, i)),
        ],
        out_shape=[
            jax.ShapeDtypeStruct((F, BS), jnp.float32),
            jax.ShapeDtypeStruct((F, BS), jnp.int32),
        ],
    )(x, idx)


def _sc_gather(table, idx_flat):
    D = table.shape[1]
    N = idx_flat.size
    info = plsc.get_sparse_core_info()
    NC, NS = info.num_cores, info.num_subcores
    NW = NC * NS
    C = 128
    CPW = N // (NW * C)
    idx3d = idx_flat.reshape(NW, CPW, C)
    mesh = plsc.VectorSubcoreMesh(core_axis_name="c", subcore_axis_name="s")

    @functools.partial(
        pl.kernel,
        mesh=mesh,
        out_type=jax.ShapeDtypeStruct((N, D), jnp.float32),
        scratch_types=[
            pltpu.VMEM((CPW, C), jnp.int32),
            pltpu.VMEM((C, D), jnp.float32),
            pltpu.SemaphoreType.DMA,
        ],
        compiler_params=pltpu.CompilerParams(use_tc_tiling_on_sc=False),
    )
    def k(table_hbm, idx_hbm, out_hbm, idx_v, rows_v, sem):
        wid = lax.axis_index("s") * NC + lax.axis_index("c")
        cbase = wid * CPW
        pltpu.sync_copy(idx_hbm.at[wid], idx_v)

        def body(c, carry):
            pltpu.async_copy(table_hbm.at[idx_v.at[c]], rows_v, sem).wait()
            pltpu.sync_copy(rows_v, out_hbm.at[pl.ds((cbase + c) * C, C)])
            return carry

        lax.fori_loop(0, CPW, body, 0)

    return k(table, idx3d)


def _tc_compute(G3, x_pair, idx_pair, phase64, TB2=512):
    F, BSH, _ = G3.shape
    HH = phase64.shape[2] // 2

    def body(g_ref, x_ref, i_ref, p_ref, o_ref, acc_ref, cnt_ref):
        f = pl.program_id(1)
        nf = pl.num_programs(1)

        @pl.when(f == 0)
        def _():
            acc_ref[...] = jnp.zeros_like(acc_ref)
            cnt_ref[...] = jnp.zeros_like(cnt_ref)

        g = g_ref[0]
        xp = x_ref[0]
        ip = i_ref[0]
        p = p_ref[0]

        valid = jnp.logical_and(ip != PADDING_IDX, ip != WAVE_IDX)
        cnt_ref[...] += valid.astype(jnp.float32)

        ef = jnp.concatenate([g[:, 0:HH], g[:, 2 * HH:3 * HH]], axis=1)
        w = jnp.concatenate([g[:, HH:2 * HH], g[:, 3 * HH:4 * HH]], axis=1)
        xe = jnp.broadcast_to(xp[:, 0:1], (TB2, HH))
        xo = jnp.broadcast_to(xp[:, 1:2], (TB2, HH))
        xb = jnp.concatenate([xe, xo], axis=1)
        d = xb * ef + p
        D = jnp.concatenate([d, d - (jnp.pi / 2)], axis=1)
        w2 = jnp.concatenate([w, w], axis=1)
        acc_ref[...] += _fast_cos(D) * w2

        @pl.when(f == nf - 1)
        def _():
            acc = acc_ref[...]
            vm = lax.rsqrt(2.0 * cnt_ref[...])
            vme = jnp.broadcast_to(vm[:, 0:1], (TB2, HH))
            vmo = jnp.broadcast_to(vm[:, 1:2], (TB2, HH))
            o_ref[...] = jnp.concatenate(
                [
                    acc[:, 0:HH] * vme,
                    acc[:, 2 * HH:3 * HH] * vme,
                    acc[:, HH:2 * HH] * vmo,
                    acc[:, 3 * HH:4 * HH] * vmo,
                ],
                axis=1,
            )

    return pl.pallas_call(
        body,
        grid=(BSH // TB2, F),
        in_specs=[
            pl.BlockSpec((1, TB2, 4 * HH), lambda i, f: (f, i, 0)),
            pl.BlockSpec((1, TB2, 2), lambda i, f: (f, i, 0)),
            pl.BlockSpec((1, TB2, 2), lambda i, f: (f, i, 0)),
            pl.BlockSpec((1, 1, 2 * HH), lambda i, f: (f, 0, 0)),
        ],
        out_specs=pl.BlockSpec((TB2, 4 * HH), lambda i, f: (i, 0)),
        out_shape=jax.ShapeDtypeStruct((BSH, 4 * HH), jnp.float32),
        scratch_shapes=[
            pltpu.VMEM((TB2, 4 * HH), jnp.float32),
            pltpu.VMEM((TB2, 2), jnp.float32),
        ],
    )(G3, x_pair, idx_pair, phase64)


def kernel(inputs, inputsType, emb_freq, emb_w, emb_phase):
    B, S, F = inputs.shape
    V, HH = emb_freq.shape
    BS = B * S
    N = BS * F

    xT, idxT = _transpose_xi(
        inputs.reshape(BS, F), inputsType.astype(jnp.int32).reshape(BS, F)
    )
    x_pair = xT.reshape(F, BS // 2, 2)
    idx_pair = idxT.reshape(F, BS // 2, 2)
    phase = emb_phase[1:F + 1]
    phase64 = jnp.concatenate([phase, phase], axis=1).reshape(F, 1, 2 * HH)

    table = _prep_table(emb_freq, emb_w)
    G = _sc_gather(table, idxT.reshape(-1))
    G3 = G.reshape(F, BS // 2, 4 * HH)
    out_pair = _tc_compute(G3, x_pair, idx_pair, phase64)
    return out_pair.reshape(B, S, 2 * HH)


# speedup vs baseline: 11.8224x; 1.1720x over previous
import functools

import jax
import jax.numpy as jnp
from jax import lax
from jax.experimental import pallas as pl
from jax.experimental.pallas import tpu as pltpu
from jax.experimental.pallas import tpu_sc as plsc

PADDING_IDX = 0
WAVE_IDX = -1

_INV2PI = 0.15915494309189535
_COS_C = (
    0.9999999922898433,
    -19.739205553483565,
    64.93917219630283,
    -85.45116501824774,
    60.17622317114795,
    -26.000498056834612,
    6.575565932039546,
)


def _fast_cos(d):
    q = d * _INV2PI
    n = lax.round(q, lax.RoundingMethod.TO_NEAREST_EVEN)
    t = q - n
    u = t * t
    r = jnp.float32(_COS_C[6])
    for c in reversed(_COS_C[:6]):
        r = r * u + jnp.float32(c)
    return r


def _prep_table(emb_freq, emb_w):
    V, HH = emb_freq.shape
    RB = 2000

    def body(f_ref, w_ref, o_ref):
        o_ref[...] = jnp.concatenate([jnp.exp(f_ref[...]), w_ref[...]], axis=1)

    return pl.pallas_call(
        body,
        grid=(V // RB,),
        in_specs=[
            pl.BlockSpec((RB, HH), lambda i: (i, 0)),
            pl.BlockSpec((RB, HH), lambda i: (i, 0)),
        ],
        out_specs=pl.BlockSpec((RB, 2 * HH), lambda i: (i, 0)),
        out_shape=jax.ShapeDtypeStruct((V, 2 * HH), jnp.float32),
    )(emb_freq, emb_w)


def _transpose_xi(x, idx, TBt=512):
    BS, F = x.shape

    def body(x_ref, i_ref, xo_ref, io_ref):
        xo_ref[...] = x_ref[...].T
        io_ref[...] = i_ref[...].T

    return pl.pallas_call(
        body,
        grid=(BS // TBt,),
        in_specs=[
            pl.BlockSpec((TBt, F), lambda i: (i, 0)),
            pl.BlockSpec((TBt, F), lambda i: (i, 0)),
        ],
        out_specs=[
            pl.BlockSpec((F, TBt), lambda i: (0, i)),
            pl.BlockSpec((F, TBt), lambda i: (0, i)),
        ],
        out_shape=[
            jax.ShapeDtypeStruct((F, BS), jnp.float32),
            jax.ShapeDtypeStruct((F, BS), jnp.int32),
        ],
    )(x, idx)


def _sc_gather(table, idx_flat):
    D = table.shape[1]
    N = idx_flat.size
    info = plsc.get_sparse_core_info()
    NC, NS = info.num_cores, info.num_subcores
    NW = NC * NS
    C = 128
    CPW = N // (NW * C)
    idx3d = idx_flat.reshape(NW, CPW, C)
    mesh = plsc.VectorSubcoreMesh(core_axis_name="c", subcore_axis_name="s")

    @functools.partial(
        pl.kernel,
        mesh=mesh,
        out_type=jax.ShapeDtypeStruct((N, D), jnp.float32),
        scratch_types=[
            pltpu.VMEM((CPW, C), jnp.int32),
            pltpu.VMEM((C, D), jnp.float32),
            pltpu.SemaphoreType.DMA,
        ],
        compiler_params=pltpu.CompilerParams(use_tc_tiling_on_sc=False),
    )
    def k(table_hbm, idx_hbm, out_hbm, idx_v, rows_v, sem):
        wid = lax.axis_index("s") * NC + lax.axis_index("c")
        cbase = wid * CPW
        pltpu.sync_copy(idx_hbm.at[wid], idx_v)

        def body(c, carry):
            pltpu.async_copy(table_hbm.at[idx_v.at[c]], rows_v, sem).wait()
            pltpu.sync_copy(rows_v, out_hbm.at[pl.ds((cbase + c) * C, C)])
            return carry

        lax.fori_loop(0, CPW, body, 0)

    return k(table, idx3d)


def _tc_compute(G3, x_pair, idx_pair, phase64, TB2=1024):
    F, BSH, _ = G3.shape
    HH = phase64.shape[2] // 2

    def body(g_ref, x_ref, i_ref, p_ref, o_ref, acc_ref, cnt_ref):
        f = pl.program_id(1)
        nf = pl.num_programs(1)

        @pl.when(f == 0)
        def _():
            acc_ref[...] = jnp.zeros_like(acc_ref)
            cnt_ref[...] = jnp.zeros_like(cnt_ref)

        g = g_ref[0]
        xp = x_ref[0]
        ip = i_ref[0]
        p = p_ref[0]

        valid = jnp.logical_and(ip != PADDING_IDX, ip != WAVE_IDX)
        cnt_ref[...] += valid.astype(jnp.float32)

        ef = jnp.concatenate([g[:, 0:HH], g[:, 2 * HH:3 * HH]], axis=1)
        w = jnp.concatenate([g[:, HH:2 * HH], g[:, 3 * HH:4 * HH]], axis=1)
        xe = jnp.broadcast_to(xp[:, 0:1], (TB2, HH))
        xo = jnp.broadcast_to(xp[:, 1:2], (TB2, HH))
        xb = jnp.concatenate([xe, xo], axis=1)
        d = xb * ef + p
        D = jnp.concatenate([d, d - (jnp.pi / 2)], axis=1)
        w2 = jnp.concatenate([w, w], axis=1)
        acc_ref[...] += _fast_cos(D) * w2

        @pl.when(f == nf - 1)
        def _():
            acc = acc_ref[...]
            vm = lax.rsqrt(2.0 * cnt_ref[...])
            vme = jnp.broadcast_to(vm[:, 0:1], (TB2, HH))
            vmo = jnp.broadcast_to(vm[:, 1:2], (TB2, HH))
            o_ref[...] = jnp.concatenate(
                [
                    acc[:, 0:HH] * vme,
                    acc[:, 2 * HH:3 * HH] * vme,
                    acc[:, HH:2 * HH] * vmo,
                    acc[:, 3 * HH:4 * HH] * vmo,
                ],
                axis=1,
            )

    return pl.pallas_call(
        body,
        grid=(BSH // TB2, F),
        in_specs=[
            pl.BlockSpec((1, TB2, 4 * HH), lambda i, f: (f, i, 0)),
            pl.BlockSpec((1, TB2, 2), lambda i, f: (f, i, 0)),
            pl.BlockSpec((1, TB2, 2), lambda i, f: (f, i, 0)),
            pl.BlockSpec((1, 1, 2 * HH), lambda i, f: (f, 0, 0)),
        ],
        out_specs=pl.BlockSpec((TB2, 4 * HH), lambda i, f: (i, 0)),
        out_shape=jax.ShapeDtypeStruct((BSH, 4 * HH), jnp.float32),
        scratch_shapes=[
            pltpu.VMEM((TB2, 4 * HH), jnp.float32),
            pltpu.VMEM((TB2, 2), jnp.float32),
        ],
    )(G3, x_pair, idx_pair, phase64)


def kernel(inputs, inputsType, emb_freq, emb_w, emb_phase):
    B, S, F = inputs.shape
    V, HH = emb_freq.shape
    BS = B * S
    N = BS * F

    xT, idxT = _transpose_xi(
        inputs.reshape(BS, F), inputsType.astype(jnp.int32).reshape(BS, F)
    )
    x_pair = xT.reshape(F, BS // 2, 2)
    idx_pair = idxT.reshape(F, BS // 2, 2)
    phase = emb_phase[1:F + 1]
    phase64 = jnp.concatenate([phase, phase], axis=1).reshape(F, 1, 2 * HH)

    table = _prep_table(emb_freq, emb_w)
    G = _sc_gather(table, idxT.reshape(-1))
    G3 = G.reshape(F, BS // 2, 4 * HH)
    out_pair = _tc_compute(G3, x_pair, idx_pair, phase64)
    return out_pair.reshape(B, S, 2 * HH)


# speedup vs baseline: 12.2934x; 1.0398x over previous
import functools

import jax
import jax.numpy as jnp
from jax import lax
from jax.experimental import pallas as pl
from jax.experimental.pallas import tpu as pltpu
from jax.experimental.pallas import tpu_sc as plsc

PADDING_IDX = 0
WAVE_IDX = -1

_INV2PI = 0.15915494309189535
_COS_C = (
    0.9999999922898433,
    -19.739205553483565,
    64.93917219630283,
    -85.45116501824774,
    60.17622317114795,
    -26.000498056834612,
    6.575565932039546,
)


def _fast_cos(d):
    q = d * _INV2PI
    n = lax.round(q, lax.RoundingMethod.TO_NEAREST_EVEN)
    t = q - n
    u = t * t
    r = jnp.float32(_COS_C[6])
    for c in reversed(_COS_C[:6]):
        r = r * u + jnp.float32(c)
    return r


def _prep_table(emb_freq, emb_w):
    V, HH = emb_freq.shape
    RB = 2000

    def body(f_ref, w_ref, o_ref):
        o_ref[...] = jnp.concatenate([jnp.exp(f_ref[...]), w_ref[...]], axis=1)

    return pl.pallas_call(
        body,
        grid=(V // RB,),
        in_specs=[
            pl.BlockSpec((RB, HH), lambda i: (i, 0)),
            pl.BlockSpec((RB, HH), lambda i: (i, 0)),
        ],
        out_specs=pl.BlockSpec((RB, 2 * HH), lambda i: (i, 0)),
        out_shape=jax.ShapeDtypeStruct((V, 2 * HH), jnp.float32),
    )(emb_freq, emb_w)


def _transpose_xi(x, idx, TBt=512):
    BS, F = x.shape

    def body(x_ref, i_ref, xo_ref, io_ref):
        xo_ref[...] = x_ref[...].T
        io_ref[...] = i_ref[...].T

    return pl.pallas_call(
        body,
        grid=(BS // TBt,),
        in_specs=[
            pl.BlockSpec((TBt, F), lambda i: (i, 0)),
            pl.BlockSpec((TBt, F), lambda i: (i, 0)),
        ],
        out_specs=[
            pl.BlockSpec((F, TBt), lambda i: (0, i)),
            pl.BlockSpec((F, TBt), lambda i: (0, i)),
        ],
        out_shape=[
            jax.ShapeDtypeStruct((F, BS), jnp.float32),
            jax.ShapeDtypeStruct((F, BS), jnp.int32),
        ],
    )(x, idx)


def _sc_gather(table, idx_flat):
    D = table.shape[1]
    N = idx_flat.size
    info = plsc.get_sparse_core_info()
    NC, NS = info.num_cores, info.num_subcores
    NW = NC * NS
    C = 128
    CPW = N // (NW * C)
    idx3d = idx_flat.reshape(NW, CPW, C)
    mesh = plsc.VectorSubcoreMesh(core_axis_name="c", subcore_axis_name="s")

    @functools.partial(
        pl.kernel,
        mesh=mesh,
        out_type=jax.ShapeDtypeStruct((N, D), jnp.float32),
        scratch_types=[
            pltpu.VMEM((CPW, C), jnp.int32),
            pltpu.VMEM((C, D), jnp.float32),
            pltpu.SemaphoreType.DMA,
        ],
        compiler_params=pltpu.CompilerParams(use_tc_tiling_on_sc=False),
    )
    def k(table_hbm, idx_hbm, out_hbm, idx_v, rows_v, sem):
        wid = lax.axis_index("s") * NC + lax.axis_index("c")
        cbase = wid * CPW
        pltpu.sync_copy(idx_hbm.at[wid], idx_v)

        def body(c, carry):
            pltpu.async_copy(table_hbm.at[idx_v.at[c]], rows_v, sem).wait()
            pltpu.sync_copy(rows_v, out_hbm.at[pl.ds((cbase + c) * C, C)])
            return carry

        lax.fori_loop(0, CPW, body, 0)

    return k(table, idx3d)


def _tc_compute(G3, x_pair, idx_pair, phase64, TB2=1600):
    F, BSH, _ = G3.shape
    HH = phase64.shape[2] // 2

    def body(g_ref, x_ref, i_ref, p_ref, o_ref, acc_ref, cnt_ref):
        f = pl.program_id(1)
        nf = pl.num_programs(1)

        @pl.when(f == 0)
        def _():
            acc_ref[...] = jnp.zeros_like(acc_ref)
            cnt_ref[...] = jnp.zeros_like(cnt_ref)

        g = g_ref[0]
        xp = x_ref[0]
        ip = i_ref[0]
        p = p_ref[0]

        valid = jnp.logical_and(ip != PADDING_IDX, ip != WAVE_IDX)
        cnt_ref[...] += valid.astype(jnp.float32)

        ef = jnp.concatenate([g[:, 0:HH], g[:, 2 * HH:3 * HH]], axis=1)
        w = jnp.concatenate([g[:, HH:2 * HH], g[:, 3 * HH:4 * HH]], axis=1)
        xe = jnp.broadcast_to(xp[:, 0:1], (TB2, HH))
        xo = jnp.broadcast_to(xp[:, 1:2], (TB2, HH))
        xb = jnp.concatenate([xe, xo], axis=1)
        d = xb * ef + p
        D = jnp.concatenate([d, d - (jnp.pi / 2)], axis=1)
        w2 = jnp.concatenate([w, w], axis=1)
        acc_ref[...] += _fast_cos(D) * w2

        @pl.when(f == nf - 1)
        def _():
            acc = acc_ref[...]
            vm = lax.rsqrt(2.0 * cnt_ref[...])
            vme = jnp.broadcast_to(vm[:, 0:1], (TB2, HH))
            vmo = jnp.broadcast_to(vm[:, 1:2], (TB2, HH))
            o_ref[...] = jnp.concatenate(
                [
                    acc[:, 0:HH] * vme,
                    acc[:, 2 * HH:3 * HH] * vme,
                    acc[:, HH:2 * HH] * vmo,
                    acc[:, 3 * HH:4 * HH] * vmo,
                ],
                axis=1,
            )

    return pl.pallas_call(
        body,
        grid=(BSH // TB2, F),
        in_specs=[
            pl.BlockSpec((1, TB2, 4 * HH), lambda i, f: (f, i, 0)),
            pl.BlockSpec((1, TB2, 2), lambda i, f: (f, i, 0)),
            pl.BlockSpec((1, TB2, 2), lambda i, f: (f, i, 0)),
            pl.BlockSpec((1, 1, 2 * HH), lambda i, f: (f, 0, 0)),
        ],
        out_specs=pl.BlockSpec((TB2, 4 * HH), lambda i, f: (i, 0)),
        out_shape=jax.ShapeDtypeStruct((BSH, 4 * HH), jnp.float32),
        scratch_shapes=[
            pltpu.VMEM((TB2, 4 * HH), jnp.float32),
            pltpu.VMEM((TB2, 2), jnp.float32),
        ],
    )(G3, x_pair, idx_pair, phase64)


def kernel(inputs, inputsType, emb_freq, emb_w, emb_phase):
    B, S, F = inputs.shape
    V, HH = emb_freq.shape
    BS = B * S
    N = BS * F

    xT, idxT = _transpose_xi(
        inputs.reshape(BS, F), inputsType.astype(jnp.int32).reshape(BS, F)
    )
    x_pair = xT.reshape(F, BS // 2, 2)
    idx_pair = idxT.reshape(F, BS // 2, 2)
    phase = emb_phase[1:F + 1]
    phase64 = jnp.concatenate([phase, phase], axis=1).reshape(F, 1, 2 * HH)

    table = _prep_table(emb_freq, emb_w)
    G = _sc_gather(table, idxT.reshape(-1))
    G3 = G.reshape(F, BS // 2, 4 * HH)
    out_pair = _tc_compute(G3, x_pair, idx_pair, phase64)
    return out_pair.reshape(B, S, 2 * HH)
